# Initial kernel scaffold; baseline (speedup 1.0000x reference)
import jax
import jax.numpy as jnp
from jax.experimental import pallas as pl


def kernel(x, edge_index, edge_weight, W0, W_adj, W_rec, W1, W2):
    raise NotImplementedError("write your pallas kernel here")


# speedup vs baseline: 6.2591x; 6.2591x over previous
import functools

import jax
import jax.numpy as jnp
from jax import lax
from jax.experimental import pallas as pl
from jax.experimental.pallas import tpu as pltpu
from jax.experimental.pallas import tpu_sc as plsc

NW = 32
CHUNK = 128
LANES = 16

_SPLAT_DNUMS = lax.GatherDimensionNumbers(
    offset_dims=(), collapsed_slice_dims=(0,), start_index_map=(0,))


def _splat(vec, lane):
    idx = jnp.full((LANES, 1), lane, jnp.int32)
    return lax.gather(vec, idx, _SPLAT_DNUMS, slice_sizes=(1,),
                      mode=lax.GatherScatterMode.PROMISE_IN_BOUNDS)


@functools.lru_cache(maxsize=None)
def _make_sc_spmm(n, c_width, nchunk):
    mesh = plsc.VectorSubcoreMesh(core_axis_name="c", subcore_axis_name="s")
    main_rows = (n // 16) & ~7
    io_rows = main_rows // 3 if main_rows % 3 == 0 else main_rows
    n_io = main_rows // io_rows
    rem_start = 16 * main_rows
    nrem_blocks = (n - rem_start) // 8

    @functools.partial(
        pl.kernel,
        out_type=jax.ShapeDtypeStruct((2 * n, c_width), jnp.float32),
        mesh=mesh,
        scratch_types=[
            pltpu.VMEM((nchunk, CHUNK), jnp.int32),
            pltpu.VMEM((nchunk, CHUNK), jnp.int32),
            pltpu.VMEM((nchunk, CHUNK), jnp.float32),
            pltpu.VMEM((CHUNK, c_width), jnp.float32),
            pltpu.VMEM((io_rows, c_width), jnp.float32),
            pltpu.VMEM_SHARED((n, c_width), jnp.float32),
            pltpu.SemaphoreType.DMA,
        ],
        compiler_params=pltpu.CompilerParams(use_tc_tiling_on_sc=False),
    )
    def spmm(t_hbm, cols_hbm, rows_hbm, w_hbm, out_hbm,
             colbuf, rowbuf, wbuf, gbuf, iobuf, acc, sem):
        cid = lax.axis_index("c")
        sid = lax.axis_index("s")
        wid = sid * 2 + cid

        pltpu.sync_copy(cols_hbm.at[wid], colbuf)
        pltpu.sync_copy(rows_hbm.at[wid], rowbuf)
        pltpu.sync_copy(w_hbm.at[wid], wbuf)

        zero = jnp.zeros((LANES,), jnp.float32)

        def zero_body(r, carry):
            for h in range(c_width // LANES):
                iobuf[r, pl.ds(h * LANES, LANES)] = zero
            return carry

        lax.fori_loop(0, io_rows, zero_body, 0)
        for i in range(n_io):
            pltpu.sync_copy(
                iobuf, acc.at[pl.ds(sid * main_rows + i * io_rows, io_rows)])

        @pl.when(sid < nrem_blocks)
        def _zero_rem():
            pltpu.sync_copy(iobuf.at[pl.ds(0, 8)],
                            acc.at[pl.ds(rem_start + sid * 8, 8)])

        plsc.subcore_barrier()

        def chunk_body(j, carry):
            pltpu.async_copy(t_hbm.at[colbuf.at[j]], gbuf, sem).wait()
            for jj in range(CHUNK // LANES):
                wv = wbuf[j, pl.ds(jj * LANES, LANES)]
                for l in range(LANES):
                    e = jj * LANES + l
                    ws = _splat(wv, l)
                    for h in range(c_width // LANES):
                        g = gbuf[e, pl.ds(h * LANES, LANES)]
                        gbuf[e, pl.ds(h * LANES, LANES)] = g * ws
            pltpu.sync_copy(gbuf, acc.at[rowbuf.at[j]], add=True)
            return carry

        lax.fori_loop(0, nchunk, chunk_body, 0)
        plsc.subcore_barrier()

        for i in range(n_io):
            s = sid * main_rows + i * io_rows
            pltpu.sync_copy(acc.at[pl.ds(s, io_rows)], iobuf)
            pltpu.sync_copy(iobuf, out_hbm.at[pl.ds(cid * n + s, io_rows)])

        @pl.when(sid < nrem_blocks)
        def _write_rem():
            s = rem_start + sid * 8
            pltpu.sync_copy(acc.at[pl.ds(s, 8)], iobuf.at[pl.ds(0, 8)])
            pltpu.sync_copy(iobuf.at[pl.ds(0, 8)],
                            out_hbm.at[pl.ds(cid * n + s, 8)])

    return spmm


def _prep_body(x_ref, w0_ref, wadj_ref, out_ref):
    out_ref[:, :w0_ref.shape[1]] = jnp.dot(
        x_ref[...], w0_ref[...], preferred_element_type=jnp.float32)
    out_ref[:, w0_ref.shape[1]:] = wadj_ref[...]


def _mid1_body(p0a_ref, p0b_ref, w1_ref, out_ref):
    s = p0a_ref[...] + p0b_ref[...]
    h = s.shape[1] // 2
    h1 = jax.nn.relu(s[:, :h]) + jax.nn.relu(s[:, h:])
    out_ref[...] = jnp.dot(h1, w1_ref[...], preferred_element_type=jnp.float32)


def _adjrec_body(pa_ref, pb_ref, wrec_ref, out_ref):
    h = pa_ref.shape[1] // 2
    hadj = jax.nn.relu(pa_ref[:, h:] + pb_ref[:, h:])
    out_ref[...] = jax.nn.relu(
        jnp.dot(hadj, wrec_ref[...], preferred_element_type=jnp.float32))


def _mid2_body(p1a_ref, p1b_ref, p0a_ref, p0b_ref, w2_ref, out_ref):
    h = p0a_ref.shape[1] // 2
    hadj = jax.nn.relu(p0a_ref[:, h:] + p0b_ref[:, h:])
    h2 = jax.nn.relu(p1a_ref[...] + p1b_ref[...]) + hadj
    out_ref[...] = jnp.dot(h2, w2_ref[...], preferred_element_type=jnp.float32)


def _zzt_body(zra_ref, zrb_ref, zca_ref, zcb_ref, out_ref):
    zr = zra_ref[...] + zrb_ref[...]
    zc = zca_ref[...] + zcb_ref[...]
    out_ref[...] = lax.dot_general(
        zr, zc, (((1,), (1,)), ((), ())), preferred_element_type=jnp.float32)


def _edge_split(edge_index, edge_weight):
    e = edge_index.shape[1]
    ept = -(-e // NW)
    nchunk = -(-ept // CHUNK)
    epad = NW * nchunk * CHUNK - e
    rows3 = jnp.pad(edge_index[0], (0, epad)).reshape(NW, nchunk, CHUNK)
    cols3 = jnp.pad(edge_index[1], (0, epad)).reshape(NW, nchunk, CHUNK)
    w3 = jnp.pad(edge_weight, (0, epad)).reshape(NW, nchunk, CHUNK)
    return cols3, rows3, w3, nchunk


def _tc_prep(x, W0, W_adj):
    n, f = x.shape
    h = W0.shape[1]
    br = 1000
    gr = n // br
    return pl.pallas_call(
        _prep_body,
        grid=(gr,),
        in_specs=[
            pl.BlockSpec((br, f), lambda i: (i, 0)),
            pl.BlockSpec((f, h), lambda i: (0, 0)),
            pl.BlockSpec((br, h), lambda i: (i, 0)),
        ],
        out_specs=pl.BlockSpec((br, 2 * h), lambda i: (i, 0)),
        out_shape=jax.ShapeDtypeStruct((n, 2 * h), jnp.float32),
    )(x, W0, W_adj)


def _tc_mid1(p0, W1):
    n = p0.shape[0] // 2
    h = W1.shape[0]
    br = 1000
    gr = n // br
    return pl.pallas_call(
        _mid1_body,
        grid=(gr,),
        in_specs=[
            pl.BlockSpec((br, 2 * h), lambda i: (i, 0)),
            pl.BlockSpec((br, 2 * h), lambda i: (i + gr, 0)),
            pl.BlockSpec((h, h), lambda i: (0, 0)),
        ],
        out_specs=pl.BlockSpec((br, h), lambda i: (i, 0)),
        out_shape=jax.ShapeDtypeStruct((n, h), jnp.float32),
    )(p0, p0, W1)


def _tc_adjrec(p0, W_rec):
    n = p0.shape[0] // 2
    h = W_rec.shape[0]
    br = 400
    gr = n // br
    return pl.pallas_call(
        _adjrec_body,
        grid=(gr,),
        in_specs=[
            pl.BlockSpec((br, 2 * h), lambda i: (i, 0)),
            pl.BlockSpec((br, 2 * h), lambda i: (i + gr, 0)),
            pl.BlockSpec((h, n), lambda i: (0, 0)),
        ],
        out_specs=pl.BlockSpec((br, n), lambda i: (i, 0)),
        out_shape=jax.ShapeDtypeStruct((n, n), jnp.float32),
    )(p0, p0, W_rec)


def _tc_mid2(p1, p0, W2):
    n = p1.shape[0] // 2
    h = W2.shape[0]
    d = W2.shape[1]
    br = 1000
    gr = n // br
    return pl.pallas_call(
        _mid2_body,
        grid=(gr,),
        in_specs=[
            pl.BlockSpec((br, h), lambda i: (i, 0)),
            pl.BlockSpec((br, h), lambda i: (i + gr, 0)),
            pl.BlockSpec((br, 2 * h), lambda i: (i, 0)),
            pl.BlockSpec((br, 2 * h), lambda i: (i + gr, 0)),
            pl.BlockSpec((h, d), lambda i: (0, 0)),
        ],
        out_specs=pl.BlockSpec((br, d), lambda i: (i, 0)),
        out_shape=jax.ShapeDtypeStruct((n, d), jnp.float32),
    )(p1, p1, p0, p0, W2)


def _tc_zzt(p2):
    n = p2.shape[0] // 2
    d = p2.shape[1]
    br = 400
    gr = n // br
    return pl.pallas_call(
        _zzt_body,
        grid=(gr,),
        in_specs=[
            pl.BlockSpec((br, d), lambda i: (i, 0)),
            pl.BlockSpec((br, d), lambda i: (i + gr, 0)),
            pl.BlockSpec((n, d), lambda i: (0, 0)),
            pl.BlockSpec((n, d), lambda i: (1, 0)),
        ],
        out_specs=pl.BlockSpec((br, n), lambda i: (i, 0)),
        out_shape=jax.ShapeDtypeStruct((n, n), jnp.float32),
    )(p2, p2, p2, p2)


def kernel(x, edge_index, edge_weight, W0, W_adj, W_rec, W1, W2):
    n = x.shape[0]
    h = W0.shape[1]
    d = W2.shape[1]

    cols3, rows3, w3, nchunk = _edge_split(edge_index, edge_weight)

    tcat = _tc_prep(x, W0, W_adj)
    p0 = _make_sc_spmm(n, 2 * h, nchunk)(tcat, cols3, rows3, w3)
    hw1 = _tc_mid1(p0, W1)
    adj_rec = _tc_adjrec(p0, W_rec)
    p1 = _make_sc_spmm(n, h, nchunk)(hw1, cols3, rows3, w3)
    hw2 = _tc_mid2(p1, p0, W2)
    p2 = _make_sc_spmm(n, d, nchunk)(hw2, cols3, rows3, w3)
    zzt = _tc_zzt(p2)

    return (jnp.reshape(zzt, (-1,)), adj_rec)
